# baseline (speedup 1.0000x reference)
import functools

import numpy as np
import jax
import jax.numpy as jnp
from jax import lax
from jax.experimental import pallas as pl
from jax.experimental.pallas import tpu as pltpu
from jax.experimental.pallas import tpu_sc as plsc

_B = 1024
_LPOS = 20
_LSESS = 50
_NCLASS = 100000
_NEG_CAND = 256

_NC = 2
_NS = 16
_NW = _NC * _NS

_CBLK = 1024
_NCB = 98
_NCPAD = _NCB * _CBLK


def _tf2x32(k0, k1, x0, x1):
    rot = [[13, 15, 26, 6], [17, 29, 16, 24]]
    ks = [np.uint32(k0), np.uint32(k1)]
    ks.append(np.uint32(ks[0] ^ ks[1] ^ np.uint32(0x1BD11BDA)))
    x0 = (x0 + ks[0]).astype(np.uint32)
    x1 = (x1 + ks[1]).astype(np.uint32)

    def rotl(v, d):
        return ((v << np.uint32(d)) | (v >> np.uint32(32 - d))).astype(
            np.uint32)

    for r in range(5):
        for d in rot[r % 2]:
            x0 = (x0 + x1).astype(np.uint32)
            x1 = rotl(x1, d)
            x1 = (x1 ^ x0).astype(np.uint32)
        x0 = (x0 + ks[(r + 1) % 3]).astype(np.uint32)
        x1 = (x1 + ks[(r + 2) % 3] + np.uint32(r + 1)).astype(np.uint32)
    return x0, x1


def _tf_pair(key, n):
    return _tf2x32(key[0], key[1], np.zeros(n, np.uint32),
                   np.arange(n, dtype=np.uint32))


def _cand_table():
    with np.errstate(over="ignore"):
        b1, b2 = _tf_pair(np.array([0, 42], np.uint32), _B)
        keys = np.stack([b1, b2], axis=1)
        span = np.uint32(_NCLASS)
        mult = np.uint32(np.uint32(2 ** 16) % span)
        mult = np.uint32((mult * mult) % span)
        out = np.empty((_B, _NEG_CAND), np.int32)
        for i in range(_B):
            s1, s2 = _tf_pair(keys[i], 2)
            hi1, hi2 = _tf_pair(np.array([s1[0], s2[0]], np.uint32),
                                _NEG_CAND)
            lo1, lo2 = _tf_pair(np.array([s1[1], s2[1]], np.uint32),
                                _NEG_CAND)
            higher = (hi1 ^ hi2).astype(np.uint32)
            lower = (lo1 ^ lo2).astype(np.uint32)
            off = ((higher % span) * mult + (lower % span)).astype(np.uint32)
            out[i] = (off % span).astype(np.int32)
    return out


_CAND = _cand_table()


def _stage_a_body(labels_ref, sessions_ref, cand_ref, neg_ref):
    labels = labels_ref[...]
    sessions = sessions_ref[...]
    cand = cand_ref[...]
    bad = jnp.zeros((_B, _NEG_CAND), dtype=jnp.bool_)
    for j in range(_LPOS):
        bad = bad | (cand == labels[:, j][:, None])
    for j in range(_LSESS):
        bad = bad | (cand == sessions[:, j][:, None])
    kiota = lax.broadcasted_iota(jnp.int32, (_B, _NEG_CAND), 1)
    km = jnp.where(bad, _NEG_CAND, kiota)
    first = jnp.min(km, axis=1)
    first = jnp.where(first == _NEG_CAND, 0, first)
    onehot = kiota == first[:, None]
    neg = jnp.sum(jnp.where(onehot, cand, 0), axis=1)
    neg_ref[...] = neg


def _stage_a(labels, sessions, cand):
    return pl.pallas_call(
        _stage_a_body,
        out_shape=jax.ShapeDtypeStruct((_B,), jnp.int32),
    )(labels, sessions, cand)


def _stage_b_body(x_ref, out_ref):
    b = pl.program_id(0)
    x = x_ref[...]
    col = b * _CBLK + lax.broadcasted_iota(jnp.int32, (_B, _CBLK), 1)
    valid = col < _NCLASS
    xs = jnp.where(valid, x, 0.0)
    p = jax.nn.sigmoid(xs)
    e = -jnp.log(1.0 - p + 1e-10)
    contrib = jnp.where(valid, e, 0.0)
    out_ref[...] = jnp.sum(contrib, axis=0)[None, None, :]


def _stage_b(outputs):
    return pl.pallas_call(
        _stage_b_body,
        grid=(_NCB,),
        in_specs=[pl.BlockSpec((_B, _CBLK), lambda b: (0, b))],
        out_specs=pl.BlockSpec((1, 1, _CBLK), lambda b: (b, 0, 0)),
        out_shape=jax.ShapeDtypeStruct((_NCB, 1, _CBLK), jnp.float32),
    )(outputs)


_PER_W = _B // _NW


def _stage_c_body(colsum_hbm, neg_hbm, out_hbm, idx_v, vals_v, sem):
    wid = lax.axis_index("s") * _NC + lax.axis_index("c")
    base = wid * _PER_W
    pltpu.sync_copy(neg_hbm.at[pl.ds(base, _PER_W)], idx_v)
    pltpu.async_copy(colsum_hbm.at[idx_v], vals_v, sem).wait()
    pltpu.sync_copy(vals_v, out_hbm.at[pl.ds(base, _PER_W)])


def _stage_c(colsum1d, neg):
    mesh = plsc.VectorSubcoreMesh(core_axis_name="c", subcore_axis_name="s")
    k = functools.partial(
        pl.kernel,
        mesh=mesh,
        out_type=jax.ShapeDtypeStruct((_B,), jnp.float32),
        scratch_types=[
            pltpu.VMEM((_PER_W,), jnp.int32),
            pltpu.VMEM((_PER_W,), jnp.float32),
            pltpu.SemaphoreType.DMA,
        ],
    )(_stage_c_body)
    return k(colsum1d, neg)


def _stage_d_body(lab_ref, *refs):
    in_refs = refs[:_LPOS]
    acc_ref = refs[_LPOS]
    i = pl.program_id(0)

    @pl.when(i == 0)
    def _init():
        acc_ref[...] = jnp.zeros((1, 128), jnp.float32)

    liota = lax.broadcasted_iota(jnp.int32, (8, 128), 1)
    riota = lax.broadcasted_iota(jnp.int32, (8, 128), 0)
    sub = i % 8
    l1 = lax.broadcasted_iota(jnp.int32, (1, 128), 1)
    vec = jnp.zeros((1, 128), jnp.float32)
    for j in range(_LPOS):
        lane = lab_ref[i * _LPOS + j] % 128
        xj = in_refs[j][...]
        sel = (liota == lane) & (riota == sub)
        val = jnp.sum(jnp.where(sel, xj, 0.0))
        vec = jnp.where(l1 == j, val, vec)
    mask = l1 < _LPOS
    p = jax.nn.sigmoid(vec)
    e = -jnp.log(p + 1e-10)
    contrib = jnp.sum(jnp.where(mask, e, 0.0))
    acc_ref[...] = acc_ref[...] + contrib


def _stage_d(outputs, labels_flat):
    def mk_spec(j):
        return pl.BlockSpec(
            (8, 128),
            lambda i, lab, j=j: (i // 8, lab[i * _LPOS + j] // 128))

    grid_spec = pltpu.PrefetchScalarGridSpec(
        num_scalar_prefetch=1,
        grid=(_B,),
        in_specs=[mk_spec(j) for j in range(_LPOS)],
        out_specs=pl.BlockSpec((1, 128), lambda i, lab: (0, 0)),
    )
    return pl.pallas_call(
        _stage_d_body,
        grid_spec=grid_spec,
        out_shape=jax.ShapeDtypeStruct((1, 128), jnp.float32),
    )(labels_flat, *([outputs] * _LPOS))


def _stage_e_body(negvals_ref, posacc_ref, out_ref):
    neg_sum = jnp.sum(negvals_ref[...])
    pos_sum = posacc_ref[...][0, 0]
    loss = neg_sum / _B + pos_sum / (_B * _LPOS)
    out_ref[...] = jnp.reshape(loss, (1, 1))


def _stage_e(negvals, posacc):
    return pl.pallas_call(
        _stage_e_body,
        out_shape=jax.ShapeDtypeStruct((1, 1), jnp.float32),
    )(negvals, posacc)


def kernel(outputs, labels, sessions):
    cand = jnp.asarray(_CAND)
    neg = _stage_a(labels, sessions, cand)
    colsum = _stage_b(outputs).reshape(_NCPAD)
    negvals = _stage_c(colsum, neg)
    posacc = _stage_d(outputs, labels.reshape(-1))
    loss = _stage_e(negvals.reshape(8, 128), posacc)
    return loss.reshape(())


# speedup vs baseline: 1.8133x; 1.8133x over previous
import functools

import numpy as np
import jax
import jax.numpy as jnp
from jax import lax
from jax.experimental import pallas as pl
from jax.experimental.pallas import tpu as pltpu
from jax.experimental.pallas import tpu_sc as plsc

_B = 1024
_LPOS = 20
_LSESS = 50
_NCLASS = 100000
_NEG_CAND = 256

_NC = 2
_NS = 16
_L = 16
_NW = _NC * _NS

_POS_PAD = 32
_N_NEG = _B * _B
_N_POS = _B * _POS_PAD
_N_TOT = _N_NEG + _N_POS
_CHUNK = 128
_N_CHUNKS = _N_TOT // _CHUNK
_CHUNKS_PER_W = _N_CHUNKS // _NW


def _tf2x32(k0, k1, x0, x1):
    rot = [[13, 15, 26, 6], [17, 29, 16, 24]]
    ks = [np.uint32(k0), np.uint32(k1)]
    ks.append(np.uint32(ks[0] ^ ks[1] ^ np.uint32(0x1BD11BDA)))
    x0 = (x0 + ks[0]).astype(np.uint32)
    x1 = (x1 + ks[1]).astype(np.uint32)

    def rotl(v, d):
        return ((v << np.uint32(d)) | (v >> np.uint32(32 - d))).astype(
            np.uint32)

    for r in range(5):
        for d in rot[r % 2]:
            x0 = (x0 + x1).astype(np.uint32)
            x1 = rotl(x1, d)
            x1 = (x1 ^ x0).astype(np.uint32)
        x0 = (x0 + ks[(r + 1) % 3]).astype(np.uint32)
        x1 = (x1 + ks[(r + 2) % 3] + np.uint32(r + 1)).astype(np.uint32)
    return x0, x1


def _tf_pair(key, n):
    return _tf2x32(key[0], key[1], np.zeros(n, np.uint32),
                   np.arange(n, dtype=np.uint32))


def _cand_table():
    with np.errstate(over="ignore"):
        b1, b2 = _tf_pair(np.array([0, 42], np.uint32), _B)
        keys = np.stack([b1, b2], axis=1)
        span = np.uint32(_NCLASS)
        mult = np.uint32(np.uint32(2 ** 16) % span)
        mult = np.uint32((mult * mult) % span)
        out = np.empty((_B, _NEG_CAND), np.int32)
        for i in range(_B):
            s1, s2 = _tf_pair(keys[i], 2)
            hi1, hi2 = _tf_pair(np.array([s1[0], s2[0]], np.uint32),
                                _NEG_CAND)
            lo1, lo2 = _tf_pair(np.array([s1[1], s2[1]], np.uint32),
                                _NEG_CAND)
            higher = (hi1 ^ hi2).astype(np.uint32)
            lower = (lo1 ^ lo2).astype(np.uint32)
            off = ((higher % span) * mult + (lower % span)).astype(np.uint32)
            out[i] = (off % span).astype(np.int32)
    return out


_CAND = _cand_table()


def _stage1_body(labels_ref, sessions_ref, cand_ref, negidx_ref, posidx_ref):
    labels = labels_ref[...]
    sessions = sessions_ref[...]
    cand = cand_ref[...]

    bad = jnp.zeros((_B, _NEG_CAND), dtype=jnp.bool_)
    for j in range(_LPOS):
        bad = bad | (cand == labels[:, j][:, None])
    for j in range(_LSESS):
        bad = bad | (cand == sessions[:, j][:, None])

    kiota = lax.broadcasted_iota(jnp.int32, (_B, _NEG_CAND), 1)
    km = jnp.where(bad, _NEG_CAND, kiota)
    first = jnp.min(km, axis=1)
    first = jnp.where(first == _NEG_CAND, 0, first)
    onehot = kiota == first[:, None]
    neg = jnp.sum(jnp.where(onehot, cand, 0), axis=1)

    riota = lax.broadcasted_iota(jnp.int32, (_B, _B), 0)
    negidx_ref[...] = riota * _NCLASS + neg[None, :]

    labels_pad = jnp.concatenate(
        [labels, jnp.zeros((_B, _POS_PAD - _LPOS), jnp.int32)], axis=1)
    riota2 = lax.broadcasted_iota(jnp.int32, (_B, _POS_PAD), 0)
    posidx_ref[...] = riota2 * _NCLASS + labels_pad


def _stage1(labels, sessions, cand):
    return pl.pallas_call(
        _stage1_body,
        out_shape=(
            jax.ShapeDtypeStruct((_B, _B), jnp.int32),
            jax.ShapeDtypeStruct((_B, _POS_PAD), jnp.int32),
        ),
    )(labels, sessions, cand)


_UNROLL = 8
_N_OUTER = _CHUNKS_PER_W // _UNROLL


def _stage2_body(table_hbm, idx_hbm, out_hbm, idx_v, vals_v, sem):
    wid = lax.axis_index("s") * _NC + lax.axis_index("c")
    row0 = wid * _CHUNKS_PER_W

    pltpu.sync_copy(idx_hbm.at[pl.ds(row0, _CHUNKS_PER_W)], idx_v)

    def fire(c, carry):
        for j in range(_UNROLL):
            r = c * _UNROLL + j
            pltpu.async_copy(table_hbm.at[idx_v.at[r]], vals_v.at[r], sem)
        return carry

    lax.fori_loop(0, _N_OUTER, fire, 0)

    pltpu.make_async_copy(
        out_hbm.at[pl.ds(row0, _CHUNKS_PER_W)], vals_v, sem).wait()

    pltpu.sync_copy(vals_v, out_hbm.at[pl.ds(row0, _CHUNKS_PER_W)])


def _stage2(table, idx_all):
    mesh = plsc.VectorSubcoreMesh(core_axis_name="c", subcore_axis_name="s")
    k = functools.partial(
        pl.kernel,
        mesh=mesh,
        out_type=jax.ShapeDtypeStruct((_N_CHUNKS, _CHUNK), jnp.float32),
        scratch_types=[
            pltpu.VMEM((_CHUNKS_PER_W, _CHUNK), jnp.int32),
            pltpu.VMEM((_CHUNKS_PER_W, _CHUNK), jnp.float32),
            pltpu.SemaphoreType.DMA,
        ],
    )(_stage2_body)
    return k(table, idx_all)


def _stage3_body(scores_ref, out_ref):
    s = scores_ref[...]
    sneg = s[: _N_NEG // 128, :]
    spos = s[_N_NEG // 128:, :]
    neg_prob = jax.nn.sigmoid(sneg)
    neg_e = -jnp.log(1.0 - neg_prob + 1e-10)
    neg_sum = jnp.sum(neg_e)
    col = lax.broadcasted_iota(jnp.int32, (_N_POS // 128, 128), 1)
    valid = (col % _POS_PAD) < _LPOS
    pos_prob = jax.nn.sigmoid(spos)
    pos_e = -jnp.log(pos_prob + 1e-10)
    pos_sum = jnp.sum(jnp.where(valid, pos_e, 0.0))
    loss = neg_sum / _B + pos_sum / (_B * _LPOS)
    out_ref[...] = jnp.reshape(loss, (1, 1))


def _stage3(scores):
    return pl.pallas_call(
        _stage3_body,
        out_shape=jax.ShapeDtypeStruct((1, 1), jnp.float32),
    )(scores)


def kernel(outputs, labels, sessions):
    cand = jnp.asarray(_CAND)
    negidx, posidx = _stage1(labels, sessions, cand)
    idx_all = jnp.concatenate(
        [negidx.reshape(-1), posidx.reshape(-1)]).reshape(_N_CHUNKS, _CHUNK)
    table = outputs.reshape(_B * _NCLASS)
    scores = _stage2(table, idx_all)
    loss = _stage3(scores)
    return loss.reshape(())
